# Initial kernel scaffold; baseline (speedup 1.0000x reference)
import jax
import jax.numpy as jnp
from jax.experimental import pallas as pl


def kernel(data, A, x0, logdx, y0, logdy, logderiv):
    raise NotImplementedError("write your pallas kernel here")


# speedup vs baseline: 2000.5871x; 2000.5871x over previous
import functools

import jax
import jax.numpy as jnp
from jax.experimental import pallas as pl

_N_BLOCK = 1024
_PREC = jax.lax.Precision.DEFAULT


def _take(tab, gi):
    return jnp.take_along_axis(tab, gi, axis=1, mode="promise_in_bounds")


def _gather2(ta, tb, ii):
    v0 = _take(ta, jnp.minimum(ii, 127))
    v1 = _take(tb, jnp.maximum(ii - 128, 0))
    return jnp.where(ii < 128, v0, v1)


def _spline_body(M, refs):
    (data_ref, a_ref, xxa_ref, xxb_ref, xk0a_ref, xk0b_ref, winva_ref,
     winvb_ref, yk0a_ref, yk0b_ref, dya_ref, dyb_ref, d0a_ref, d0b_ref,
     ata_ref, atb_ref, b_ref, out_ref, logj_ref) = refs
    blk = data_ref[:, :]
    a = a_ref[:, :]
    x = jax.lax.dot_general(a, blk, (((0,), (1,)), ((), ())),
                            preferred_element_type=jnp.float32,
                            precision=_PREC)

    xxa = xxa_ref[:, :]
    xxb = xxb_ref[:, :]
    idx = jnp.where(xxa[:, 127:128] < x, 128, 0).astype(jnp.int32)
    for p in (64, 32, 16, 8, 4, 2, 1):
        gi = jnp.minimum(idx + (p - 1), M - 1)
        vals = _gather2(xxa, xxb, gi)
        ok = (vals < x) & (idx + p <= M)
        idx = jnp.where(ok, idx + p, idx)

    ii = jnp.clip(idx, 1, M - 1) - 1
    xk0 = _gather2(xk0a_ref[:, :], xk0b_ref[:, :], ii)
    winv = _gather2(winva_ref[:, :], winvb_ref[:, :], ii)
    yk0 = _gather2(yk0a_ref[:, :], yk0b_ref[:, :], ii)
    dyv = _gather2(dya_ref[:, :], dyb_ref[:, :], ii)
    d0 = _gather2(d0a_ref[:, :], d0b_ref[:, :], ii)
    atv = _gather2(ata_ref[:, :], atb_ref[:, :], ii)

    sv = dyv * winv
    xi = jnp.clip((x - xk0) * winv, 0.0, 1.0)
    xi2 = xi * xi
    xi1 = xi - xi2
    denom = sv + atv * xi1
    rden = 1.0 / denom
    numy = (sv - d0) * xi2 + d0 * xi
    y_in = yk0 + dyv * numy * rden
    numd = atv * xi2 + 2.0 * (sv - d0) * xi + d0
    logd_in = jnp.log(sv * sv * numd * (rden * rden))

    b = b_ref[:, :]
    xx0, xxl = b[:, 0:1], b[:, 1:2]
    yy0, yyl = b[:, 2:3], b[:, 3:4]
    dl0, dll = b[:, 4:5], b[:, 5:6]
    ld0, ldl = b[:, 6:7], b[:, 7:8]
    left = idx == 0
    right = idx == M
    y = jnp.where(left, yy0 + (x - xx0) * dl0,
                  jnp.where(right, yyl + (x - xxl) * dll, y_in))
    logd = jnp.where(left, jnp.broadcast_to(ld0, x.shape),
                     jnp.where(right, jnp.broadcast_to(ldl, x.shape),
                               logd_in))

    logj_ref[...] = jnp.sum(logd, axis=0)
    out_ref[:, :] = blk + jax.lax.dot_general(
        y - x, a, (((0,), (1,)), ((), ())),
        preferred_element_type=jnp.float32, precision=_PREC)


def _body(M, *refs):
    _spline_body(M, refs)


def kernel(data, A, x0, logdx, y0, logdy, logderiv):
    N, D = data.shape
    K = A.shape[1]
    M = logderiv.shape[1]

    xx = jnp.concatenate([x0, x0 + jnp.cumsum(jnp.exp(logdx), axis=1)], axis=1)
    yy = jnp.concatenate([y0, y0 + jnp.cumsum(jnp.exp(logdy), axis=1)], axis=1)
    delta = jnp.exp(logderiv)
    w = xx[:, 1:] - xx[:, :-1]
    winv = 1.0 / w
    dy = yy[:, 1:] - yy[:, :-1]
    s = dy * winv
    at = delta[:, :-1] + delta[:, 1:] - 2.0 * s

    def halves(t):
        ta = t[:, :128]
        tb = t[:, 128:]
        tb = jnp.pad(tb, ((0, 0), (0, 128 - tb.shape[1])))
        return ta, tb

    tables = []
    for t in (xx, xx[:, :-1], winv, yy[:, :-1], dy, delta[:, :-1], at):
        tables.extend(halves(t))
    bpack = jnp.stack([xx[:, 0], xx[:, M - 1], yy[:, 0], yy[:, M - 1],
                       delta[:, 0], delta[:, M - 1],
                       logderiv[:, 0], logderiv[:, M - 1]], axis=1)

    Bn = _N_BLOCK
    grid = (pl.cdiv(N, Bn),)
    const = lambda shape: pl.BlockSpec(shape, lambda i: (0, 0))
    out, logj = pl.pallas_call(
        functools.partial(_body, M),
        grid=grid,
        in_specs=[
            pl.BlockSpec((Bn, D), lambda i: (i, 0)),
            const((D, K)),
        ] + [const((K, 128)) for _ in range(14)] + [const((K, 8))],
        out_specs=[
            pl.BlockSpec((Bn, D), lambda i: (i, 0)),
            pl.BlockSpec((Bn,), lambda i: (i,)),
        ],
        out_shape=[
            jax.ShapeDtypeStruct((N, D), jnp.float32),
            jax.ShapeDtypeStruct((N,), jnp.float32),
        ],
    )(data, A, *tables, bpack)
    return (out, logj)


# speedup vs baseline: 3146.5350x; 1.5728x over previous
import functools

import jax
import jax.numpy as jnp
from jax.experimental import pallas as pl

_N_BLOCK = 1024
_LANE_TILE = 256
_INTERLEAVE = 1
_PREC = jax.lax.Precision.DEFAULT


def _take(tab, gi):
    return jnp.take_along_axis(tab, gi, axis=1, mode="promise_in_bounds")


def _spline_body(M, refs):
    (data_ref, a_ref, ca_ref, xo_ref, xk0a_ref, xk0b_ref, winva_ref,
     winvb_ref, yk0a_ref, yk0b_ref, dya_ref, dyb_ref, d0a_ref, d0b_ref,
     ata_ref, atb_ref, b_ref, out_ref, logj_ref) = refs
    a = a_ref[:, :]
    ca = ca_ref[:, :]
    xo = xo_ref[:, :]
    xk0a, xk0b = xk0a_ref[:, :], xk0b_ref[:, :]
    winva, winvb = winva_ref[:, :], winvb_ref[:, :]
    yk0a, yk0b = yk0a_ref[:, :], yk0b_ref[:, :]
    dya, dyb = dya_ref[:, :], dyb_ref[:, :]
    d0a, d0b = d0a_ref[:, :], d0b_ref[:, :]
    ata, atb = ata_ref[:, :], atb_ref[:, :]
    b = b_ref[:, :]
    xx0, xxl = b[:, 0:1], b[:, 1:2]
    db0, dbl = b[:, 4:5], b[:, 5:6]
    W = _LANE_TILE
    NC = _N_BLOCK // W
    blks = [data_ref[W * c:W * c + W, :] for c in range(NC)]
    xs = [jax.lax.dot_general(a, blk, (((0,), (1,)), ((), ())),
                              preferred_element_type=jnp.float32,
                              precision=_PREC) for blk in blks]
    for c in range(NC):
        c0 = W * c
        x = xs[c]
        terms = [jnp.where(ca[:, 0:1] < x, 1, 0).astype(jnp.int32)] + [
            jnp.where(ca[:, 8 * t:8 * t + 1] < x, 8, 0).astype(jnp.int32)
            for t in range(1, 13)]
        while len(terms) > 1:
            terms = ([terms[i] + terms[i + 1]
                      for i in range(0, len(terms) - 1, 2)]
                     + ([terms[-1]] if len(terms) % 2 else []))
        jc = terms[0]
        for p in (4, 2, 1):
            vals = _take(ca, jc + (p - 1))
            jc = jc + jnp.where(vals < x, p, 0)
        idx = jnp.maximum(2 * jc - 1, 0) + (_take(xo, jc) < x).astype(
            jnp.int32)
        ii = jnp.clip(idx - 1, 0, M - 2)

        lo = ii < 128
        i0 = jnp.minimum(ii, 127)
        i1 = jnp.maximum(ii - 128, 0)
        g2 = lambda ta, tb: jnp.where(lo, _take(ta, i0), _take(tb, i1))
        xk0 = g2(xk0a, xk0b)
        winv = g2(winva, winvb)
        yk0 = g2(yk0a, yk0b)
        dyv = g2(dya, dyb)
        d0 = g2(d0a, d0b)
        atv = g2(ata, atb)

        sv = dyv * winv
        xi = jnp.clip((x - xk0) * winv, 0.0, 1.0)
        xi2 = xi * xi
        xi1 = xi - xi2
        denom = sv + atv * xi1
        rden = 1.0 / denom
        t1 = sv - d0
        t3 = t1 * xi
        u = t3 + d0
        numy = u * xi
        y_in = yk0 + dyv * numy * rden
        numd = atv * xi2 + (u + t3)
        tsr = sv * rden
        logd = jnp.log(tsr * tsr * numd)
        e_lo = jnp.minimum(x - xx0, 0.0)
        e_hi = jnp.maximum(x - xxl, 0.0)
        diff = (y_in - x) + e_lo * db0 + e_hi * dbl

        logj_ref[c0:c0 + W] = jnp.sum(logd, axis=0)
        out_ref[c0:c0 + W, :] = blks[c] + jax.lax.dot_general(
            diff, a, (((0,), (1,)), ((), ())),
            preferred_element_type=jnp.float32, precision=_PREC)


def _body(M, *refs):
    _spline_body(M, refs)


def kernel(data, A, x0, logdx, y0, logdy, logderiv):
    N, D = data.shape
    K = A.shape[1]
    M = logderiv.shape[1]

    xx = jnp.concatenate([x0, x0 + jnp.cumsum(jnp.exp(logdx), axis=1)], axis=1)
    yy = jnp.concatenate([y0, y0 + jnp.cumsum(jnp.exp(logdy), axis=1)], axis=1)
    delta = jnp.exp(logderiv)
    w = xx[:, 1:] - xx[:, :-1]
    winv = 1.0 / w
    dy = yy[:, 1:] - yy[:, :-1]
    s = dy * winv
    at = delta[:, :-1] + delta[:, 1:] - 2.0 * s

    def pad128(t, val=0.0):
        return jnp.pad(t, ((0, 0), (0, 128 - t.shape[1])),
                       constant_values=val)

    def halves(t):
        return t[:, :128], pad128(t[:, 128:])

    MC = (M + 1) // 2
    jj = jnp.maximum(2 * jnp.arange(MC + 1) - 1, 0)
    tables = [pad128(xx[:, ::2], jnp.inf), pad128(xx[:, jj], jnp.inf)]
    for t in (xx[:, :-1], winv, yy[:, :-1], dy, delta[:, :-1], at):
        tables.extend(halves(t))
    bpack = jnp.stack([xx[:, 0], xx[:, M - 1], yy[:, 0], yy[:, M - 1],
                       delta[:, 0], delta[:, M - 1],
                       logderiv[:, 0], logderiv[:, M - 1]], axis=1)

    Bn = _N_BLOCK
    grid = (pl.cdiv(N, Bn),)
    const = lambda shape: pl.BlockSpec(shape, lambda i: (0, 0))
    out, logj = pl.pallas_call(
        functools.partial(_body, M),
        grid=grid,
        in_specs=[
            pl.BlockSpec((Bn, D), lambda i: (i, 0)),
            const((D, K)),
        ] + [const((K, 128)) for _ in range(14)] + [const((K, 8))],
        out_specs=[
            pl.BlockSpec((Bn, D), lambda i: (i, 0)),
            pl.BlockSpec((Bn,), lambda i: (i,)),
        ],
        out_shape=[
            jax.ShapeDtypeStruct((N, D), jnp.float32),
            jax.ShapeDtypeStruct((N,), jnp.float32),
        ],
    )(data, A, *tables, bpack)
    return (out, logj)


# speedup vs baseline: 3443.1088x; 1.0943x over previous
import functools

import jax
import jax.numpy as jnp
from jax.experimental import pallas as pl

_N_BLOCK = 2048
_LANE_TILE = 256
_INTERLEAVE = 1
_PREC = jax.lax.Precision.DEFAULT


def _take(tab, gi):
    return jnp.take_along_axis(tab, gi, axis=1, mode="promise_in_bounds")


def _spline_body(M, refs):
    (data_ref, a_ref, ca_ref, xo_ref, xk0a_ref, xk0b_ref, winva_ref,
     winvb_ref, yk0a_ref, yk0b_ref, dya_ref, dyb_ref, d0a_ref, d0b_ref,
     ata_ref, atb_ref, b_ref, out_ref, logj_ref) = refs
    a = a_ref[:, :]
    ca = ca_ref[:, :]
    xo = xo_ref[:, :]
    xk0a, xk0b = xk0a_ref[:, :], xk0b_ref[:, :]
    winva, winvb = winva_ref[:, :], winvb_ref[:, :]
    yk0a, yk0b = yk0a_ref[:, :], yk0b_ref[:, :]
    dya, dyb = dya_ref[:, :], dyb_ref[:, :]
    d0a, d0b = d0a_ref[:, :], d0b_ref[:, :]
    ata, atb = ata_ref[:, :], atb_ref[:, :]
    b = b_ref[:, :]
    xx0, xxl = b[:, 0:1], b[:, 1:2]
    db0, dbl = b[:, 4:5], b[:, 5:6]
    W = _LANE_TILE
    NC = _N_BLOCK // W
    blks = [data_ref[W * c:W * c + W, :] for c in range(NC)]
    xs = [jax.lax.dot_general(a, blk, (((0,), (1,)), ((), ())),
                              preferred_element_type=jnp.float32,
                              precision=_PREC) for blk in blks]
    for c in range(NC):
        c0 = W * c
        x = xs[c]
        terms = [jnp.where(ca[:, 0:1] < x, 1, 0).astype(jnp.int32)] + [
            jnp.where(ca[:, 4 * t:4 * t + 1] < x, 4, 0).astype(jnp.int32)
            for t in range(1, 25)]
        while len(terms) > 1:
            terms = ([terms[i] + terms[i + 1]
                      for i in range(0, len(terms) - 1, 2)]
                     + ([terms[-1]] if len(terms) % 2 else []))
        jc = terms[0]
        for p in (2, 1):
            vals = _take(ca, jc + (p - 1))
            jc = jc + jnp.where(vals < x, p, 0)
        idx = jnp.maximum(2 * jc - 1, 0) + (_take(xo, jc) < x).astype(
            jnp.int32)
        ii = jnp.clip(idx - 1, 0, M - 2)

        lo = ii < 128
        i0 = jnp.minimum(ii, 127)
        i1 = jnp.maximum(ii - 128, 0)
        g2 = lambda ta, tb: jnp.where(lo, _take(ta, i0), _take(tb, i1))
        xk0 = g2(xk0a, xk0b)
        winv = g2(winva, winvb)
        xi = jnp.clip((x - xk0) * winv, 0.0, 1.0)
        xi2 = xi * xi
        xi1 = xi - xi2
        e_lo = jnp.minimum(x - xx0, 0.0)
        e_hi = jnp.maximum(x - xxl, 0.0)
        tail = e_lo * db0 + e_hi * dbl
        dyv = g2(dya, dyb)
        d0 = g2(d0a, d0b)
        sv = dyv * winv
        t1 = sv - d0
        t3 = t1 * xi
        u = t3 + d0
        numy = u * xi
        atv = g2(ata, atb)
        denom = sv + atv * xi1
        rden = 1.0 / denom
        numd = atv * xi2 + (u + t3)
        tsr = sv * rden
        logd = jnp.log(tsr * tsr * numd)
        yk0 = g2(yk0a, yk0b)
        y_in = yk0 + dyv * numy * rden
        diff = (y_in - x) + tail

        logj_ref[c0:c0 + W] = jnp.sum(logd, axis=0)
        out_ref[c0:c0 + W, :] = blks[c] + jax.lax.dot_general(
            diff, a, (((0,), (1,)), ((), ())),
            preferred_element_type=jnp.float32, precision=_PREC)


def _body(M, *refs):
    _spline_body(M, refs)


def kernel(data, A, x0, logdx, y0, logdy, logderiv):
    N, D = data.shape
    K = A.shape[1]
    M = logderiv.shape[1]

    xx = jnp.concatenate([x0, x0 + jnp.cumsum(jnp.exp(logdx), axis=1)], axis=1)
    yy = jnp.concatenate([y0, y0 + jnp.cumsum(jnp.exp(logdy), axis=1)], axis=1)
    delta = jnp.exp(logderiv)
    w = xx[:, 1:] - xx[:, :-1]
    winv = 1.0 / w
    dy = yy[:, 1:] - yy[:, :-1]
    s = dy * winv
    at = delta[:, :-1] + delta[:, 1:] - 2.0 * s

    def pad128(t, val=0.0):
        return jnp.pad(t, ((0, 0), (0, 128 - t.shape[1])),
                       constant_values=val)

    def halves(t):
        return t[:, :128], pad128(t[:, 128:])

    MC = (M + 1) // 2
    jj = jnp.maximum(2 * jnp.arange(MC + 1) - 1, 0)
    tables = [pad128(xx[:, ::2], jnp.inf), pad128(xx[:, jj], jnp.inf)]
    for t in (xx[:, :-1], winv, yy[:, :-1], dy, delta[:, :-1], at):
        tables.extend(halves(t))
    bpack = jnp.stack([xx[:, 0], xx[:, M - 1], yy[:, 0], yy[:, M - 1],
                       delta[:, 0], delta[:, M - 1],
                       logderiv[:, 0], logderiv[:, M - 1]], axis=1)

    Bn = _N_BLOCK
    grid = (pl.cdiv(N, Bn),)
    const = lambda shape: pl.BlockSpec(shape, lambda i: (0, 0))
    out, logj = pl.pallas_call(
        functools.partial(_body, M),
        grid=grid,
        in_specs=[
            pl.BlockSpec((Bn, D), lambda i: (i, 0)),
            const((D, K)),
        ] + [const((K, 128)) for _ in range(14)] + [const((K, 8))],
        out_specs=[
            pl.BlockSpec((Bn, D), lambda i: (i, 0)),
            pl.BlockSpec((Bn,), lambda i: (i,)),
        ],
        out_shape=[
            jax.ShapeDtypeStruct((N, D), jnp.float32),
            jax.ShapeDtypeStruct((N,), jnp.float32),
        ],
    )(data, A, *tables, bpack)
    return (out, logj)


# speedup vs baseline: 3582.5079x; 1.0405x over previous
import functools

import jax
import jax.numpy as jnp
from jax.experimental import pallas as pl

_N_BLOCK = 4096
_LANE_TILE = 256
_INTERLEAVE = 1
_PREC = jax.lax.Precision.DEFAULT


def _take(tab, gi):
    return jnp.take_along_axis(tab, gi, axis=1, mode="promise_in_bounds")


def _spline_body(M, refs):
    (data_ref, a_ref, ca_ref, xo_ref, xk0a_ref, xk0b_ref, winva_ref,
     winvb_ref, yk0a_ref, yk0b_ref, dya_ref, dyb_ref, d0a_ref, d0b_ref,
     ata_ref, atb_ref, b_ref, out_ref, logj_ref) = refs
    a = a_ref[:, :]
    ca = ca_ref[:, :]
    xo = xo_ref[:, :]
    xk0a, xk0b = xk0a_ref[:, :], xk0b_ref[:, :]
    winva, winvb = winva_ref[:, :], winvb_ref[:, :]
    yk0a, yk0b = yk0a_ref[:, :], yk0b_ref[:, :]
    dya, dyb = dya_ref[:, :], dyb_ref[:, :]
    d0a, d0b = d0a_ref[:, :], d0b_ref[:, :]
    ata, atb = ata_ref[:, :], atb_ref[:, :]
    b = b_ref[:, :]
    xx0, xxl = b[:, 0:1], b[:, 1:2]
    db0, dbl = b[:, 4:5], b[:, 5:6]
    W = _LANE_TILE
    NC = _N_BLOCK // W
    blks = [data_ref[W * c:W * c + W, :] for c in range(NC)]
    xs = [jax.lax.dot_general(a, blk, (((0,), (1,)), ((), ())),
                              preferred_element_type=jnp.float32,
                              precision=_PREC) for blk in blks]
    for c in range(NC):
        c0 = W * c
        x = xs[c]
        terms = [jnp.where(ca[:, 0:1] < x, 1, 0).astype(jnp.int32)] + [
            jnp.where(ca[:, 4 * t:4 * t + 1] < x, 4, 0).astype(jnp.int32)
            for t in range(1, 25)]
        while len(terms) > 1:
            terms = ([terms[i] + terms[i + 1]
                      for i in range(0, len(terms) - 1, 2)]
                     + ([terms[-1]] if len(terms) % 2 else []))
        jc = terms[0]
        for p in (2, 1):
            vals = _take(ca, jc + (p - 1))
            jc = jc + jnp.where(vals < x, p, 0)
        idx = jnp.maximum(2 * jc - 1, 0) + (_take(xo, jc) < x).astype(
            jnp.int32)
        ii = jnp.clip(idx - 1, 0, M - 2)

        lo = ii < 128
        i0 = jnp.minimum(ii, 127)
        i1 = jnp.maximum(ii - 128, 0)
        g2 = lambda ta, tb: jnp.where(lo, _take(ta, i0), _take(tb, i1))
        xk0 = g2(xk0a, xk0b)
        winv = g2(winva, winvb)
        xi = jnp.clip((x - xk0) * winv, 0.0, 1.0)
        xi2 = xi * xi
        xi1 = xi - xi2
        e_lo = jnp.minimum(x - xx0, 0.0)
        e_hi = jnp.maximum(x - xxl, 0.0)
        tail = e_lo * db0 + e_hi * dbl
        dyv = g2(dya, dyb)
        d0 = g2(d0a, d0b)
        sv = dyv * winv
        t1 = sv - d0
        t3 = t1 * xi
        u = t3 + d0
        numy = u * xi
        atv = g2(ata, atb)
        denom = sv + atv * xi1
        rden = 1.0 / denom
        numd = atv * xi2 + (u + t3)
        tsr = sv * rden
        logd = jnp.log(tsr * tsr * numd)
        yk0 = g2(yk0a, yk0b)
        y_in = yk0 + dyv * numy * rden
        diff = (y_in - x) + tail

        logj_ref[c0:c0 + W] = jnp.sum(logd, axis=0)
        out_ref[c0:c0 + W, :] = blks[c] + jax.lax.dot_general(
            diff, a, (((0,), (1,)), ((), ())),
            preferred_element_type=jnp.float32, precision=_PREC)


def _body(M, *refs):
    _spline_body(M, refs)


def kernel(data, A, x0, logdx, y0, logdy, logderiv):
    N, D = data.shape
    K = A.shape[1]
    M = logderiv.shape[1]

    xx = jnp.concatenate([x0, x0 + jnp.cumsum(jnp.exp(logdx), axis=1)], axis=1)
    yy = jnp.concatenate([y0, y0 + jnp.cumsum(jnp.exp(logdy), axis=1)], axis=1)
    delta = jnp.exp(logderiv)
    w = xx[:, 1:] - xx[:, :-1]
    winv = 1.0 / w
    dy = yy[:, 1:] - yy[:, :-1]
    s = dy * winv
    at = delta[:, :-1] + delta[:, 1:] - 2.0 * s

    def pad128(t, val=0.0):
        return jnp.pad(t, ((0, 0), (0, 128 - t.shape[1])),
                       constant_values=val)

    def halves(t):
        return t[:, :128], pad128(t[:, 128:])

    MC = (M + 1) // 2
    jj = jnp.maximum(2 * jnp.arange(MC + 1) - 1, 0)
    tables = [pad128(xx[:, ::2], jnp.inf), pad128(xx[:, jj], jnp.inf)]
    for t in (xx[:, :-1], winv, yy[:, :-1], dy, delta[:, :-1], at):
        tables.extend(halves(t))
    bpack = jnp.stack([xx[:, 0], xx[:, M - 1], yy[:, 0], yy[:, M - 1],
                       delta[:, 0], delta[:, M - 1],
                       logderiv[:, 0], logderiv[:, M - 1]], axis=1)

    Bn = _N_BLOCK
    grid = (pl.cdiv(N, Bn),)
    const = lambda shape: pl.BlockSpec(shape, lambda i: (0, 0))
    out, logj = pl.pallas_call(
        functools.partial(_body, M),
        grid=grid,
        in_specs=[
            pl.BlockSpec((Bn, D), lambda i: (i, 0)),
            const((D, K)),
        ] + [const((K, 128)) for _ in range(14)] + [const((K, 8))],
        out_specs=[
            pl.BlockSpec((Bn, D), lambda i: (i, 0)),
            pl.BlockSpec((Bn,), lambda i: (i,)),
        ],
        out_shape=[
            jax.ShapeDtypeStruct((N, D), jnp.float32),
            jax.ShapeDtypeStruct((N,), jnp.float32),
        ],
    )(data, A, *tables, bpack)
    return (out, logj)
